# baseline (speedup 1.0000x reference)
import functools

import jax
import jax.numpy as jnp
from jax import lax
from jax.experimental import pallas as pl
from jax.experimental.pallas import tpu as pltpu
from jax.experimental.pallas import tpu_sc as plsc

VOCAB = 100000
D = 200
DH = 128
DT = D - DH
DPAD = 256
BATCH = 4096
HIST = 50
HPAD = 56

NC = 2
NS = 16
NW = NC * NS
BPW = BATCH // NW
NBUF = 4


def _gather_body(idx_hbm, table_hbm, out_hbm, idx_v, *bufs_and_sems):
    tbufs = bufs_and_sems[0:NBUF]
    obufs = bufs_and_sems[NBUF:2 * NBUF]
    hsems = bufs_and_sems[2 * NBUF:3 * NBUF]
    tsems = bufs_and_sems[3 * NBUF:4 * NBUF]
    wid = lax.axis_index("s") * NC + lax.axis_index("c")
    pltpu.sync_copy(idx_hbm.at[pl.ds(wid * BPW * HPAD, BPW * HPAD)], idx_v)
    b0 = wid * BPW
    head_view = table_hbm.at[:, pl.ds(0, DH)]
    tail_view = table_hbm.at[:, pl.ds(DH, DH)]
    lane16 = lax.iota(jnp.int32, 16)
    lanes = DH + 64 + lane16
    tail8 = lane16 < (DT - 64)

    def start(bb, b):
        idx = idx_v.at[pl.ds(bb * HPAD, HIST)]
        pltpu.async_copy(head_view.at[idx], obufs[b].at[:, pl.ds(0, DH)],
                         hsems[b])
        pltpu.async_copy(tail_view.at[idx], tbufs[b], tsems[b])

    def finish(bb, b):
        idx = idx_v.at[pl.ds(bb * HPAD, HIST)]
        pltpu.make_async_copy(head_view.at[idx], obufs[b].at[:, pl.ds(0, DH)],
                              hsems[b]).wait()
        pltpu.make_async_copy(tail_view.at[idx], tbufs[b], tsems[b]).wait()
        ob = obufs[b]
        tb = tbufs[b]

        def row(r, carry):
            for k in range(4):
                ob[r, pl.ds(DH + 16 * k, 16)] = tb[r, pl.ds(16 * k, 16)]
            rows = jnp.full((16,), r, jnp.int32)
            plsc.store_scatter(ob, [rows, lanes], tb[r, pl.ds(64, 16)],
                               mask=tail8)
            return carry

        lax.fori_loop(0, HIST, row, 0)
        pltpu.sync_copy(ob, out_hbm.at[b0 + bb])

    for b in range(NBUF):
        start(b, b)

    def step(i, carry):
        g = i * NBUF
        for b in range(NBUF):
            bb = g + b
            finish(bb, b)
            start(bb + NBUF, b)
        return carry

    lax.fori_loop(0, BPW // NBUF - 1, step, 0)
    for b in range(NBUF):
        finish(BPW - NBUF + b, b)


@jax.jit
def _embedding_lookup(idx1d, table_pad):
    return pl.kernel(
        _gather_body,
        out_type=jax.ShapeDtypeStruct((BATCH, HIST, D), jnp.float32),
        mesh=plsc.VectorSubcoreMesh(core_axis_name="c", subcore_axis_name="s"),
        scratch_types=(
            [pltpu.VMEM((BPW * HPAD,), jnp.int32)]
            + [pltpu.VMEM((HIST, DH), jnp.float32)] * NBUF
            + [pltpu.VMEM((HIST, D), jnp.float32)] * NBUF
            + [pltpu.SemaphoreType.DMA] * (2 * NBUF)
        ),
        compiler_params=pltpu.CompilerParams(needs_layout_passes=False),
    )(idx1d, table_pad)


def kernel(clauses, E):
    idx = clauses.astype(jnp.int32)
    idx = jnp.pad(idx, ((0, 0), (0, HPAD - HIST)))
    idx1d = idx.reshape(BATCH * HPAD)
    e_pad = jnp.pad(E, ((0, 0), (0, DPAD - D)))
    return _embedding_lookup(idx1d, e_pad)


# speedup vs baseline: 1.6320x; 1.6320x over previous
import functools

import jax
import jax.numpy as jnp
from jax import lax
from jax.experimental import pallas as pl
from jax.experimental.pallas import tpu as pltpu
from jax.experimental.pallas import tpu_sc as plsc

VOCAB = 100000
D = 200
DH = 128
DT = D - DH
BATCH = 4096
HIST = 50
HPAD = 56

NC = 2
NS = 16
NW = NC * NS
BPW = BATCH // NW
NBUF = 4

RPW = 3136
PCHUNK = 112
PNCH = RPW // PCHUNK


def _prep_body(table_hbm, tail_hbm, i0, i1, o0, o1, is0, is1, os0, os1):
    wid = lax.axis_index("s") * NC + lax.axis_index("c")
    sw = (wid * (VOCAB // NW)) // 8 * 8
    sw = jnp.minimum(sw, VOCAB - RPW)
    ibufs = (i0, i1)
    obufs = (o0, o1)
    isems = (is0, is1)
    osems = (os0, os1)
    lane16 = lax.iota(jnp.int32, 16)
    g_lanes = DH + 64 + lane16
    s_lanes = 64 + lane16
    last8 = lane16 < (DT - 64)

    def start(c, b):
        pltpu.async_copy(table_hbm.at[pl.ds(sw + c * PCHUNK, PCHUNK)],
                         ibufs[b], isems[b])

    def finish(c, b, drain):
        r0 = sw + c * PCHUNK
        pltpu.make_async_copy(table_hbm.at[pl.ds(r0, PCHUNK)],
                              ibufs[b], isems[b]).wait()
        ib, ob = ibufs[b], obufs[b]
        if drain:
            pltpu.make_async_copy(ob, tail_hbm.at[pl.ds(0, PCHUNK)],
                                  osems[b]).wait()

        def row(r, carry):
            for k in range(4):
                ob[r, pl.ds(16 * k, 16)] = ib[r, pl.ds(DH + 16 * k, 16)]
            rows = jnp.full((16,), r, jnp.int32)
            v = plsc.load_gather(ib, [rows, g_lanes], mask=last8)
            plsc.store_scatter(ob, [rows, s_lanes], v, mask=last8)
            return carry

        lax.fori_loop(0, PCHUNK, row, 0)
        pltpu.async_copy(ob, tail_hbm.at[pl.ds(r0, PCHUNK)], osems[b])

    for b in range(2):
        start(b, b)
    for b in range(2):
        finish(b, b, drain=False)
        start(b + 2, b)

    def step(i, carry):
        g = 2 + i * 2
        for b in range(2):
            c = g + b
            finish(c, b, drain=True)
            start(c + 2, b)
        return carry

    lax.fori_loop(0, PNCH // 2 - 2, step, 0)
    for b in range(2):
        finish(PNCH - 2 + b, b, drain=True)
        pltpu.make_async_copy(obufs[b], tail_hbm.at[pl.ds(0, PCHUNK)],
                              osems[b]).wait()


def _gather_body(idx_hbm, table_hbm, tail_hbm, out_hbm, idx_v,
                 *bufs_and_sems):
    tbufs = bufs_and_sems[0:NBUF]
    obufs = bufs_and_sems[NBUF:2 * NBUF]
    hsems = bufs_and_sems[2 * NBUF:3 * NBUF]
    tsems = bufs_and_sems[3 * NBUF:4 * NBUF]
    wid = lax.axis_index("s") * NC + lax.axis_index("c")
    pltpu.sync_copy(idx_hbm.at[pl.ds(wid * BPW * HPAD, BPW * HPAD)], idx_v)
    b0 = wid * BPW
    head_view = table_hbm.at[:, pl.ds(0, DH)]
    lane16 = lax.iota(jnp.int32, 16)
    lanes = DH + 64 + lane16
    tail8 = lane16 < (DT - 64)

    def start(bb, b):
        idx = idx_v.at[pl.ds(bb * HPAD, HIST)]
        pltpu.async_copy(head_view.at[idx], obufs[b].at[:, pl.ds(0, DH)],
                         hsems[b])
        pltpu.async_copy(tail_hbm.at[idx], tbufs[b], tsems[b])

    def finish(bb, b):
        idx = idx_v.at[pl.ds(bb * HPAD, HIST)]
        pltpu.make_async_copy(head_view.at[idx], obufs[b].at[:, pl.ds(0, DH)],
                              hsems[b]).wait()
        pltpu.make_async_copy(tail_hbm.at[idx], tbufs[b], tsems[b]).wait()
        ob = obufs[b]
        tb = tbufs[b]
        for r in range(HIST):
            for k in range(4):
                ob[r, pl.ds(DH + 16 * k, 16)] = tb[r, pl.ds(16 * k, 16)]
            rows = jnp.full((16,), r, jnp.int32)
            plsc.store_scatter(ob, [rows, lanes], tb[r, pl.ds(64, 16)],
                               mask=tail8)
        pltpu.sync_copy(ob, out_hbm.at[b0 + bb])

    for b in range(NBUF):
        start(b, b)

    def step(i, carry):
        g = i * NBUF
        for b in range(NBUF):
            bb = g + b
            finish(bb, b)
            start(bb + NBUF, b)
        return carry

    lax.fori_loop(0, BPW // NBUF - 1, step, 0)
    for b in range(NBUF):
        finish(BPW - NBUF + b, b)


_SC_MESH = dict(core_axis_name="c", subcore_axis_name="s")


@jax.jit
def _embedding_lookup(idx1d, table):
    tail = pl.kernel(
        _prep_body,
        out_type=jax.ShapeDtypeStruct((VOCAB, DH), jnp.float32),
        mesh=plsc.VectorSubcoreMesh(**_SC_MESH),
        scratch_types=[
            pltpu.VMEM((PCHUNK, D), jnp.float32),
            pltpu.VMEM((PCHUNK, D), jnp.float32),
            pltpu.VMEM((PCHUNK, DH), jnp.float32),
            pltpu.VMEM((PCHUNK, DH), jnp.float32),
            pltpu.SemaphoreType.DMA,
            pltpu.SemaphoreType.DMA,
            pltpu.SemaphoreType.DMA,
            pltpu.SemaphoreType.DMA,
        ],
        compiler_params=pltpu.CompilerParams(needs_layout_passes=False),
    )(table)
    return pl.kernel(
        _gather_body,
        out_type=jax.ShapeDtypeStruct((BATCH, HIST, D), jnp.float32),
        mesh=plsc.VectorSubcoreMesh(**_SC_MESH),
        scratch_types=(
            [pltpu.VMEM((BPW * HPAD,), jnp.int32)]
            + [pltpu.VMEM((HIST, DH), jnp.float32)] * NBUF
            + [pltpu.VMEM((HIST, D), jnp.float32)] * NBUF
            + [pltpu.SemaphoreType.DMA] * (2 * NBUF)
        ),
        compiler_params=pltpu.CompilerParams(needs_layout_passes=False),
    )(idx1d, table, tail)


def kernel(clauses, E):
    idx = clauses.astype(jnp.int32)
    idx = jnp.pad(idx, ((0, 0), (0, HPAD - HIST)))
    idx1d = idx.reshape(BATCH * HPAD)
    return _embedding_lookup(idx1d, E)
